# baseline (speedup 1.0000x reference)
import functools

import jax
import jax.numpy as jnp
from jax import lax
from jax.experimental import pallas as pl
from jax.experimental.pallas import tpu as pltpu
from jax.experimental.pallas import tpu_sc as plsc

EMBED_DIM = 64
NUM_CORES = 2
NUM_SUBCORES = 16
NUM_WORKERS = NUM_CORES * NUM_SUBCORES
CHUNK = 128
NBUF = 8


def _lookup(ids_flat, w_all):
    n = ids_flat.shape[0]
    per_w = n // NUM_WORKERS
    n_chunks = per_w // CHUNK
    rounds = n_chunks // NBUF
    assert per_w * NUM_WORKERS == n and rounds * NBUF == n_chunks

    mesh = plsc.VectorSubcoreMesh(core_axis_name="c", subcore_axis_name="s")

    scratch = []
    for _ in range(NBUF):
        scratch += [
            pltpu.VMEM((CHUNK,), jnp.int32),
            pltpu.VMEM((CHUNK, EMBED_DIM), jnp.float32),
            pltpu.SemaphoreType.DMA,
            pltpu.SemaphoreType.DMA,
            pltpu.SemaphoreType.DMA,
        ]

    @functools.partial(
        pl.kernel,
        out_type=jax.ShapeDtypeStruct((n, EMBED_DIM), jnp.float32),
        mesh=mesh,
        compiler_params=pltpu.CompilerParams(
            use_tc_tiling_on_sc=False, needs_layout_passes=False),
        scratch_types=scratch,
    )
    def body(ids_hbm, w_hbm, out_hbm, *bufs):
        wid = lax.axis_index("s") * NUM_CORES + lax.axis_index("c")
        base = wid * per_w

        def slot(b):
            return bufs[b * 5:(b + 1) * 5]

        def fire_ids(b, c):
            ids_v, _, sem_ids, _, _ = slot(b)
            pltpu.async_copy(
                ids_hbm.at[pl.ds(base + c * CHUNK, CHUNK)], ids_v, sem_ids)

        def wait_ids_fire_gather(b, c):
            ids_v, rows, sem_ids, sem_g, _ = slot(b)
            pltpu.make_async_copy(
                ids_hbm.at[pl.ds(base + c * CHUNK, CHUNK)], ids_v,
                sem_ids).wait()
            pltpu.async_copy(w_hbm.at[ids_v], rows, sem_g)

        def wait_gather(b):
            ids_v, rows, _, sem_g, _ = slot(b)
            pltpu.make_async_copy(w_hbm.at[ids_v], rows, sem_g).wait()

        def fire_out(b, c):
            _, rows, _, _, sem_o = slot(b)
            pltpu.async_copy(
                rows, out_hbm.at[pl.ds(base + c * CHUNK, CHUNK)], sem_o)

        def wait_out(b, c):
            _, rows, _, _, sem_o = slot(b)
            pltpu.make_async_copy(
                rows, out_hbm.at[pl.ds(base + c * CHUNK, CHUNK)], sem_o).wait()

        LOOK = NBUF // 2
        for b in range(NBUF):
            fire_ids(b, b)
        for b in range(LOOK):
            wait_ids_fire_gather(b, b)

        def round_body(r, carry):
            for b in range(NBUF):
                c = r * NBUF + b
                wait_gather(b)
                fire_out(b, c)

                @pl.when(c + NBUF < n_chunks)
                def _():
                    fire_ids(b, c + NBUF)

                b4 = (b + LOOK) % NBUF
                c4 = c + LOOK

                @pl.when(c4 >= NBUF)
                def _():
                    wait_out(b4, c4 - NBUF)

                @pl.when(c4 < n_chunks)
                def _():
                    wait_ids_fire_gather(b4, c4)
            return carry

        lax.fori_loop(0, rounds, round_body, 0)
        for c in range(n_chunks - LOOK, n_chunks):
            wait_out(c % NBUF, c)

    return body(ids_flat, w_all)


def kernel(input_ids, W_frozen, W_trainable):
    w_all = jnp.concatenate([W_frozen, W_trainable], axis=0)
    ids_flat = input_ids.reshape(-1)
    out = _lookup(ids_flat, w_all)
    return out.reshape(input_ids.shape + (EMBED_DIM,))


# speedup vs baseline: 1.5283x; 1.5283x over previous
import functools

import jax
import jax.numpy as jnp
from jax import lax
from jax.experimental import pallas as pl
from jax.experimental.pallas import tpu as pltpu
from jax.experimental.pallas import tpu_sc as plsc

EMBED_DIM = 64
NUM_CORES = 2
NUM_SUBCORES = 16
NUM_WORKERS = NUM_CORES * NUM_SUBCORES
CHUNK = 128
NBUF = 4
ROW = 128


def _lookup(ids_flat, w_all):
    n = ids_flat.shape[0]
    per_w = n // NUM_WORKERS
    n_chunks = per_w // CHUNK
    rounds = n_chunks // NBUF
    assert per_w * NUM_WORKERS == n and rounds * NBUF == n_chunks

    mesh = plsc.VectorSubcoreMesh(core_axis_name="c", subcore_axis_name="s")

    scratch = []
    for _ in range(NBUF):
        scratch += [
            pltpu.VMEM((CHUNK,), jnp.int32),
            pltpu.VMEM((CHUNK, ROW), jnp.float32),
            pltpu.SemaphoreType.DMA,
            pltpu.SemaphoreType.DMA,
            pltpu.SemaphoreType.DMA,
        ]

    @functools.partial(
        pl.kernel,
        out_type=jax.ShapeDtypeStruct((n, ROW), jnp.float32),
        mesh=mesh,
        compiler_params=pltpu.CompilerParams(needs_layout_passes=False),
        scratch_types=scratch,
    )
    def body(ids_hbm, w_hbm, out_hbm, *bufs):
        wid = lax.axis_index("s") * NUM_CORES + lax.axis_index("c")
        base = wid * per_w

        def slot(b):
            return bufs[b * 5:(b + 1) * 5]

        def fire_ids(b, c):
            ids_v, _, sem_ids, _, _ = slot(b)
            pltpu.async_copy(
                ids_hbm.at[pl.ds(base + c * CHUNK, CHUNK)], ids_v, sem_ids)

        def wait_ids_fire_gather(b, c):
            ids_v, rows, sem_ids, sem_g, _ = slot(b)
            pltpu.make_async_copy(
                ids_hbm.at[pl.ds(base + c * CHUNK, CHUNK)], ids_v,
                sem_ids).wait()
            pltpu.async_copy(w_hbm.at[ids_v], rows, sem_g)

        def wait_gather(b):
            ids_v, rows, _, sem_g, _ = slot(b)
            pltpu.make_async_copy(w_hbm.at[ids_v], rows, sem_g).wait()

        def fire_out(b, c):
            _, rows, _, _, sem_o = slot(b)
            pltpu.async_copy(
                rows, out_hbm.at[pl.ds(base + c * CHUNK, CHUNK)], sem_o)

        def wait_out(b, c):
            _, rows, _, _, sem_o = slot(b)
            pltpu.make_async_copy(
                rows, out_hbm.at[pl.ds(base + c * CHUNK, CHUNK)], sem_o).wait()

        LOOK = NBUF // 2
        for b in range(NBUF):
            fire_ids(b, b)
        for b in range(LOOK):
            wait_ids_fire_gather(b, b)

        def round_body(r, carry):
            for b in range(NBUF):
                c = r * NBUF + b
                wait_gather(b)
                fire_out(b, c)

                @pl.when(c + NBUF < n_chunks)
                def _():
                    fire_ids(b, c + NBUF)

                b4 = (b + LOOK) % NBUF
                c4 = c + LOOK

                @pl.when(c4 >= NBUF)
                def _():
                    wait_out(b4, c4 - NBUF)

                @pl.when(c4 < n_chunks)
                def _():
                    wait_ids_fire_gather(b4, c4)
            return carry

        lax.fori_loop(0, rounds, round_body, 0)
        for c in range(n_chunks - LOOK, n_chunks):
            wait_out(c % NBUF, c)

    return body(ids_flat, w_all)


def kernel(input_ids, W_frozen, W_trainable):
    w_all = jnp.concatenate([W_frozen, W_trainable], axis=0)
    w_all = jnp.pad(w_all, ((0, 0), (0, ROW - EMBED_DIM)))
    ids_flat = input_ids.reshape(-1)
    out = _lookup(ids_flat, w_all)
    return out[:, :EMBED_DIM].reshape(input_ids.shape + (EMBED_DIM,))
